# baseline (speedup 1.0000x reference)
import functools

import jax
import jax.numpy as jnp
from jax import lax
from jax.experimental import pallas as pl
from jax.experimental.pallas import tpu as pltpu
from jax.experimental.pallas import tpu_sc as plsc

N_NODES = 10000
N_EDGES = 320000
D = 128
BN_EPS = 1e-5

NC = 2
NS = 16
NW = NC * NS
EPW = N_EDGES // NW
CH = 80
CHUNKS = EPW // CH
NBUF = 2
FULL = (CHUNKS // NBUF) * NBUF
STRIPE = 624
TAIL_BASE = NS * STRIPE
TAIL = N_NODES - TAIL_BASE
ZROWS = NBUF * CH


def _sc_aggregate(src2, dst3, feats):
    mesh = plsc.VectorSubcoreMesh(
        core_axis_name="c", subcore_axis_name="s",
        num_cores=NC, num_subcores=NS)

    @functools.partial(
        pl.kernel,
        out_type=jax.ShapeDtypeStruct((NC * N_NODES, D), jnp.float32),
        mesh=mesh,
        scratch_types=[
            pltpu.VMEM((EPW,), jnp.int32),
            pltpu.VMEM((CHUNKS, CH), jnp.int32),
            pltpu.VMEM((ZROWS, D), jnp.float32),
            pltpu.VMEM_SHARED((N_NODES, D), jnp.float32),
        ] + [pltpu.SemaphoreType.DMA] * NBUF,
        compiler_params=pltpu.CompilerParams(use_tc_tiling_on_sc=False),
    )
    def k(src_hbm, dst_hbm, feats_hbm, out_hbm,
          src_v, dst_v, rows_v, agg_sh, *sems):
        c = lax.axis_index("c")
        s = lax.axis_index("s")
        w = c * NS + s

        def buf(p):
            return rows_v.at[pl.ds(p * CH, CH)]

        def fire_gather(j, p):
            start = pl.multiple_of(j * CH, 8)
            pltpu.async_copy(
                feats_hbm.at[src_v.at[pl.ds(start, CH)]], buf(p), sems[p])

        def wait_dma(p):
            pltpu.make_async_copy(
                feats_hbm.at[pl.ds(0, CH)], buf(p), sems[p]).wait()

        def fire_scatter(j, p):
            pltpu.async_copy(
                buf(p), agg_sh.at[dst_v.at[j]], sems[p], add=True)

        zv = jnp.zeros((16,), jnp.float32)

        def zrow(i, carry):
            for q in range(D // 16):
                rows_v[i, pl.ds(q * 16, 16)] = zv
            return carry

        lax.fori_loop(0, ZROWS, zrow, 0)
        base = s * STRIPE
        for r in range(STRIPE // ZROWS):
            pltpu.sync_copy(rows_v,
                            agg_sh.at[pl.ds(base + r * ZROWS, ZROWS)])
        rem = STRIPE % ZROWS
        pltpu.sync_copy(rows_v.at[pl.ds(0, rem)],
                        agg_sh.at[pl.ds(base + STRIPE - rem, rem)])

        @pl.when(s == 0)
        def _zero_tail():
            pltpu.sync_copy(rows_v.at[pl.ds(0, TAIL)],
                            agg_sh.at[pl.ds(TAIL_BASE, TAIL)])

        pltpu.sync_copy(src_hbm.at[w], src_v)
        pltpu.sync_copy(dst_hbm.at[w], dst_v)
        plsc.subcore_barrier()

        fire_gather(0, 0)

        def outer(i, carry):
            for kk in range(NBUF):
                j = i * NBUF + kk
                pn = (kk + 1) % NBUF

                @pl.when(j >= NBUF - 1)
                def _drain():
                    wait_dma(pn)

                fire_gather(j + 1, pn)
                wait_dma(kk)
                fire_scatter(j, kk)
            return carry

        lax.fori_loop(0, FULL // NBUF - 1, outer, 0)
        for j in range(FULL - NBUF, CHUNKS):
            kk = j % NBUF
            pn = (kk + 1) % NBUF
            wait_dma(pn)
            if j + 1 < CHUNKS:
                fire_gather(j + 1, pn)
            wait_dma(kk)
            fire_scatter(j, kk)
        for kd in range(NBUF - 1):
            wait_dma((CHUNKS - 1 - kd) % NBUF)

        plsc.subcore_barrier()
        pltpu.sync_copy(
            agg_sh.at[pl.ds(s * STRIPE, STRIPE)],
            out_hbm.at[pl.ds(c * N_NODES + s * STRIPE, STRIPE)])

        @pl.when(s == 0)
        def _write_tail():
            pltpu.sync_copy(
                agg_sh.at[pl.ds(TAIL_BASE, TAIL)],
                out_hbm.at[pl.ds(c * N_NODES + TAIL_BASE, TAIL)])

    return k(src2, dst3, feats)


def _tc_tail(partials, W, b, gamma, beta):
    def body(p_ref, w_ref, b_ref, g_ref, bt_ref, o_ref):
        agg = p_ref[:N_NODES, :] + p_ref[N_NODES:, :]
        h = jnp.dot(agg, w_ref[...], preferred_element_type=jnp.float32)
        h = h + b_ref[...]
        mean = jnp.mean(h, axis=0, keepdims=True)
        ctr = h - mean
        var = jnp.mean(ctr * ctr, axis=0, keepdims=True)
        o_ref[...] = g_ref[...] * ctr * lax.rsqrt(var + BN_EPS) + bt_ref[...]

    return pl.pallas_call(
        body,
        out_shape=jax.ShapeDtypeStruct((N_NODES, D), jnp.float32),
    )(partials, W, b, gamma, beta)


def kernel(g, feats, W, b, gamma, beta):
    src2 = g[0].reshape(NW, EPW)
    dst3 = g[1].reshape(NW, CHUNKS, CH)
    partials = _sc_aggregate(src2, dst3, feats)
    return _tc_tail(partials, W, b.reshape(1, D),
                    gamma.reshape(1, D), beta.reshape(1, D))


# speedup vs baseline: 1.0732x; 1.0732x over previous
import functools

import jax
import jax.numpy as jnp
from jax import lax
from jax.experimental import pallas as pl
from jax.experimental.pallas import tpu as pltpu
from jax.experimental.pallas import tpu_sc as plsc

N_NODES = 10000
N_EDGES = 320000
D = 128
BN_EPS = 1e-5

NC = 2
NS = 16
NW = NC * NS
EPW = N_EDGES // NW
CH = 80
CHUNKS = EPW // CH
NBUF = 2
FULL = (CHUNKS // NBUF) * NBUF
STRIPE = 624
TAIL_BASE = NS * STRIPE
TAIL = N_NODES - TAIL_BASE
ZROWS = NBUF * CH


def _sc_aggregate(g4, feats):
    mesh = plsc.VectorSubcoreMesh(
        core_axis_name="c", subcore_axis_name="s",
        num_cores=NC, num_subcores=NS)

    @functools.partial(
        pl.kernel,
        out_type=jax.ShapeDtypeStruct((NC * N_NODES, D), jnp.float32),
        mesh=mesh,
        scratch_types=[
            pltpu.VMEM((CHUNKS, CH), jnp.int32),
            pltpu.VMEM((CHUNKS, CH), jnp.int32),
            pltpu.VMEM((ZROWS, D), jnp.float32),
            pltpu.VMEM_SHARED((N_NODES, D), jnp.float32),
        ] + [pltpu.SemaphoreType.DMA] * NBUF,
        compiler_params=pltpu.CompilerParams(use_tc_tiling_on_sc=False),
    )
    def k(g_hbm, feats_hbm, out_hbm,
          src_v, dst_v, rows_v, agg_sh, *sems):
        c = lax.axis_index("c")
        s = lax.axis_index("s")
        w = c * NS + s

        def buf(p):
            return rows_v.at[pl.ds(p * CH, CH)]

        def fire_gather(j, p):
            pltpu.async_copy(
                feats_hbm.at[src_v.at[j]], buf(p), sems[p])

        def wait_dma(p):
            pltpu.make_async_copy(
                feats_hbm.at[pl.ds(0, CH)], buf(p), sems[p]).wait()

        def fire_scatter(j, p):
            pltpu.async_copy(
                buf(p), agg_sh.at[dst_v.at[j]], sems[p], add=True)

        zv = jnp.zeros((16,), jnp.float32)

        def zrow(i, carry):
            for q in range(D // 16):
                rows_v[i, pl.ds(q * 16, 16)] = zv
            return carry

        lax.fori_loop(0, ZROWS, zrow, 0)
        base = s * STRIPE
        for r in range(STRIPE // ZROWS):
            pltpu.sync_copy(rows_v,
                            agg_sh.at[pl.ds(base + r * ZROWS, ZROWS)])
        rem = STRIPE % ZROWS
        pltpu.sync_copy(rows_v.at[pl.ds(0, rem)],
                        agg_sh.at[pl.ds(base + STRIPE - rem, rem)])

        @pl.when(s == 0)
        def _zero_tail():
            pltpu.sync_copy(rows_v.at[pl.ds(0, TAIL)],
                            agg_sh.at[pl.ds(TAIL_BASE, TAIL)])

        pltpu.sync_copy(g_hbm.at[0, w], src_v)
        pltpu.sync_copy(g_hbm.at[1, w], dst_v)
        plsc.subcore_barrier()

        fire_gather(0, 0)

        def outer(i, carry):
            for kk in range(NBUF):
                j = i * NBUF + kk
                pn = (kk + 1) % NBUF

                @pl.when(j >= NBUF - 1)
                def _drain():
                    wait_dma(pn)

                fire_gather(j + 1, pn)
                wait_dma(kk)
                fire_scatter(j, kk)
            return carry

        lax.fori_loop(0, FULL // NBUF - 1, outer, 0)
        for j in range(FULL - NBUF, CHUNKS):
            kk = j % NBUF
            pn = (kk + 1) % NBUF
            wait_dma(pn)
            if j + 1 < CHUNKS:
                fire_gather(j + 1, pn)
            wait_dma(kk)
            fire_scatter(j, kk)
        for kd in range(NBUF - 1):
            wait_dma((CHUNKS - 1 - kd) % NBUF)

        plsc.subcore_barrier()
        pltpu.sync_copy(
            agg_sh.at[pl.ds(s * STRIPE, STRIPE)],
            out_hbm.at[pl.ds(c * N_NODES + s * STRIPE, STRIPE)])

        @pl.when(s == 0)
        def _write_tail():
            pltpu.sync_copy(
                agg_sh.at[pl.ds(TAIL_BASE, TAIL)],
                out_hbm.at[pl.ds(c * N_NODES + TAIL_BASE, TAIL)])

    return k(g4, feats)


def _tc_tail(partials, W, b, gamma, beta):
    def body(p_ref, w_ref, b_ref, g_ref, bt_ref, o_ref):
        agg = p_ref[:N_NODES, :] + p_ref[N_NODES:, :]
        h = jnp.dot(agg, w_ref[...], preferred_element_type=jnp.float32)
        h = h + b_ref[...]
        mean = jnp.mean(h, axis=0, keepdims=True)
        ctr = h - mean
        var = jnp.mean(ctr * ctr, axis=0, keepdims=True)
        o_ref[...] = g_ref[...] * ctr * lax.rsqrt(var + BN_EPS) + bt_ref[...]

    return pl.pallas_call(
        body,
        out_shape=jax.ShapeDtypeStruct((N_NODES, D), jnp.float32),
    )(partials, W, b, gamma, beta)


def kernel(g, feats, W, b, gamma, beta):
    g4 = g.reshape(2, NW, CHUNKS, CH)
    partials = _sc_aggregate(g4, feats)
    return _tc_tail(partials, W, b.reshape(1, D),
                    gamma.reshape(1, D), beta.reshape(1, D))


# speedup vs baseline: 1.2217x; 1.1383x over previous
import functools

import jax
import jax.numpy as jnp
from jax import lax
from jax.experimental import pallas as pl
from jax.experimental.pallas import tpu as pltpu
from jax.experimental.pallas import tpu_sc as plsc

N_NODES = 10000
N_EDGES = 320000
D = 128
BN_EPS = 1e-5

NC = 2
NS = 16
NW = NC * NS
EPW = N_EDGES // NW
CH = 80
CHUNKS = EPW // CH
NBUF = 3
FULL = (CHUNKS // NBUF) * NBUF
STRIPE = 624
TAIL_BASE = NS * STRIPE
TAIL = N_NODES - TAIL_BASE
ZROWS = NBUF * CH


def _sc_aggregate(g4, feats):
    mesh = plsc.VectorSubcoreMesh(
        core_axis_name="c", subcore_axis_name="s",
        num_cores=NC, num_subcores=NS)

    @functools.partial(
        pl.kernel,
        out_type=jax.ShapeDtypeStruct((NC * N_NODES, D), jnp.float32),
        mesh=mesh,
        scratch_types=[
            pltpu.VMEM((CHUNKS, CH), jnp.int32),
            pltpu.VMEM((CHUNKS, CH), jnp.int32),
            pltpu.VMEM((ZROWS, D), jnp.float32),
            pltpu.VMEM_SHARED((N_NODES, D), jnp.float32),
        ] + [pltpu.SemaphoreType.DMA] * NBUF,
        compiler_params=pltpu.CompilerParams(use_tc_tiling_on_sc=False),
    )
    def k(g_hbm, feats_hbm, out_hbm,
          src_v, dst_v, rows_v, agg_sh, *sems):
        c = lax.axis_index("c")
        s = lax.axis_index("s")
        w = c * NS + s

        def buf(p):
            return rows_v.at[pl.ds(p * CH, CH)]

        def fire_gather(j, p):
            pltpu.async_copy(
                feats_hbm.at[src_v.at[j]], buf(p), sems[p])

        def wait_dma(p):
            pltpu.make_async_copy(
                feats_hbm.at[pl.ds(0, CH)], buf(p), sems[p]).wait()

        def fire_scatter(j, p):
            pltpu.async_copy(
                buf(p), agg_sh.at[dst_v.at[j]], sems[p], add=True)

        zv = jnp.zeros((16,), jnp.float32)

        def zrow(i, carry):
            for q in range(D // 16):
                rows_v[i, pl.ds(q * 16, 16)] = zv
            return carry

        lax.fori_loop(0, ZROWS, zrow, 0)
        base = s * STRIPE
        for r in range(STRIPE // ZROWS):
            pltpu.sync_copy(rows_v,
                            agg_sh.at[pl.ds(base + r * ZROWS, ZROWS)])
        rem = STRIPE % ZROWS
        pltpu.sync_copy(rows_v.at[pl.ds(0, rem)],
                        agg_sh.at[pl.ds(base + STRIPE - rem, rem)])

        @pl.when(s == 0)
        def _zero_tail():
            pltpu.sync_copy(rows_v.at[pl.ds(0, TAIL)],
                            agg_sh.at[pl.ds(TAIL_BASE, TAIL)])

        pltpu.sync_copy(g_hbm.at[0, w], src_v)
        pltpu.sync_copy(g_hbm.at[1, w], dst_v)
        plsc.subcore_barrier()

        fire_gather(0, 0)

        def outer(i, carry):
            for kk in range(NBUF):
                j = i * NBUF + kk
                pn = (kk + 1) % NBUF

                @pl.when(j >= NBUF - 1)
                def _drain():
                    wait_dma(pn)

                fire_gather(j + 1, pn)
                wait_dma(kk)
                fire_scatter(j, kk)
            return carry

        lax.fori_loop(0, FULL // NBUF - 1, outer, 0)
        for j in range(FULL - NBUF, CHUNKS):
            kk = j % NBUF
            pn = (kk + 1) % NBUF
            wait_dma(pn)
            if j + 1 < CHUNKS:
                fire_gather(j + 1, pn)
            wait_dma(kk)
            fire_scatter(j, kk)
        for kd in range(NBUF - 1):
            wait_dma((CHUNKS - 1 - kd) % NBUF)

        plsc.subcore_barrier()
        pltpu.sync_copy(
            agg_sh.at[pl.ds(s * STRIPE, STRIPE)],
            out_hbm.at[pl.ds(c * N_NODES + s * STRIPE, STRIPE)])

        @pl.when(s == 0)
        def _write_tail():
            pltpu.sync_copy(
                agg_sh.at[pl.ds(TAIL_BASE, TAIL)],
                out_hbm.at[pl.ds(c * N_NODES + TAIL_BASE, TAIL)])

    return k(g4, feats)


def _tc_tail(partials, W, b, gamma, beta):
    def body(p_ref, w_ref, b_ref, g_ref, bt_ref, o_ref):
        agg = p_ref[:N_NODES, :] + p_ref[N_NODES:, :]
        h = jnp.dot(agg, w_ref[...], preferred_element_type=jnp.float32)
        h = h + b_ref[...]
        mean = jnp.mean(h, axis=0, keepdims=True)
        ctr = h - mean
        var = jnp.mean(ctr * ctr, axis=0, keepdims=True)
        o_ref[...] = g_ref[...] * ctr * lax.rsqrt(var + BN_EPS) + bt_ref[...]

    return pl.pallas_call(
        body,
        out_shape=jax.ShapeDtypeStruct((N_NODES, D), jnp.float32),
    )(partials, W, b, gamma, beta)


def kernel(g, feats, W, b, gamma, beta):
    g4 = g.reshape(2, NW, CHUNKS, CH)
    partials = _sc_aggregate(g4, feats)
    return _tc_tail(partials, W, b.reshape(1, D),
                    gamma.reshape(1, D), beta.reshape(1, D))


# speedup vs baseline: 1.2438x; 1.0181x over previous
import functools

import jax
import jax.numpy as jnp
from jax import lax
from jax.experimental import pallas as pl
from jax.experimental.pallas import tpu as pltpu
from jax.experimental.pallas import tpu_sc as plsc

N_NODES = 10000
N_EDGES = 320000
D = 128
BN_EPS = 1e-5

NC = 2
NS = 16
NW = NC * NS
EPW = N_EDGES // NW
CH = 80
CHUNKS = EPW // CH
NBUF = 3
FULL = (CHUNKS // NBUF) * NBUF
STRIPE = 624
TAIL_BASE = NS * STRIPE
TAIL = N_NODES - TAIL_BASE
ZROWS = NBUF * CH


def _sc_aggregate(g4, feats):
    mesh = plsc.VectorSubcoreMesh(
        core_axis_name="c", subcore_axis_name="s",
        num_cores=NC, num_subcores=NS)

    @functools.partial(
        pl.kernel,
        out_type=jax.ShapeDtypeStruct((NC * N_NODES, D), jnp.float32),
        mesh=mesh,
        scratch_types=[
            pltpu.VMEM((CHUNKS, CH), jnp.int32),
            pltpu.VMEM((CHUNKS, CH), jnp.int32),
            pltpu.VMEM((ZROWS, D), jnp.float32),
            pltpu.VMEM_SHARED((N_NODES, D), jnp.float32),
        ] + [pltpu.SemaphoreType.DMA] * NBUF,
        compiler_params=pltpu.CompilerParams(use_tc_tiling_on_sc=False),
    )
    def k(g_hbm, feats_hbm, out_hbm,
          src_v, dst_v, rows_v, agg_sh, *sems):
        c = lax.axis_index("c")
        s = lax.axis_index("s")
        w = c * NS + s

        def buf(p):
            return rows_v.at[pl.ds(p * CH, CH)]

        def fire_gather(j, p):
            pltpu.async_copy(
                feats_hbm.at[src_v.at[j]], buf(p), sems[p])

        def wait_dma(p):
            pltpu.make_async_copy(
                feats_hbm.at[pl.ds(0, CH)], buf(p), sems[p]).wait()

        def fire_scatter(j, p):
            pltpu.async_copy(
                buf(p), agg_sh.at[dst_v.at[j]], sems[p], add=True)

        pltpu.async_copy(g_hbm.at[0, w], src_v, sems[0])
        pltpu.async_copy(g_hbm.at[1, w], dst_v, sems[1])

        zv = jnp.zeros((16,), jnp.float32)

        def zrow(i, carry):
            for q in range(D // 16):
                rows_v[i, pl.ds(q * 16, 16)] = zv
            return carry

        lax.fori_loop(0, ZROWS, zrow, 0)
        base = s * STRIPE
        for r in range(STRIPE // ZROWS):
            pltpu.sync_copy(rows_v,
                            agg_sh.at[pl.ds(base + r * ZROWS, ZROWS)])
        rem = STRIPE % ZROWS
        pltpu.sync_copy(rows_v.at[pl.ds(0, rem)],
                        agg_sh.at[pl.ds(base + STRIPE - rem, rem)])

        @pl.when(s == 0)
        def _zero_tail():
            pltpu.sync_copy(rows_v.at[pl.ds(0, TAIL)],
                            agg_sh.at[pl.ds(TAIL_BASE, TAIL)])

        pltpu.make_async_copy(g_hbm.at[0, w], src_v, sems[0]).wait()
        pltpu.make_async_copy(g_hbm.at[1, w], dst_v, sems[1]).wait()
        plsc.subcore_barrier()

        fire_gather(0, 0)

        def outer(i, carry):
            for kk in range(NBUF):
                j = i * NBUF + kk
                pn = (kk + 1) % NBUF

                @pl.when(j >= NBUF - 1)
                def _drain():
                    wait_dma(pn)

                fire_gather(j + 1, pn)
                wait_dma(kk)
                fire_scatter(j, kk)
            return carry

        lax.fori_loop(0, FULL // NBUF - 1, outer, 0)
        for j in range(FULL - NBUF, CHUNKS):
            kk = j % NBUF
            pn = (kk + 1) % NBUF
            wait_dma(pn)
            if j + 1 < CHUNKS:
                fire_gather(j + 1, pn)
            wait_dma(kk)
            fire_scatter(j, kk)
        for kd in range(NBUF - 1):
            wait_dma((CHUNKS - 1 - kd) % NBUF)

        plsc.subcore_barrier()
        pltpu.sync_copy(
            agg_sh.at[pl.ds(s * STRIPE, STRIPE)],
            out_hbm.at[pl.ds(c * N_NODES + s * STRIPE, STRIPE)])

        @pl.when(s == 0)
        def _write_tail():
            pltpu.sync_copy(
                agg_sh.at[pl.ds(TAIL_BASE, TAIL)],
                out_hbm.at[pl.ds(c * N_NODES + TAIL_BASE, TAIL)])

    return k(g4, feats)


def _tc_tail(partials, W, b, gamma, beta):
    def body(p_ref, w_ref, b_ref, g_ref, bt_ref, o_ref):
        agg = p_ref[:N_NODES, :] + p_ref[N_NODES:, :]
        h = jnp.dot(agg, w_ref[...], preferred_element_type=jnp.float32)
        h = h + b_ref[...]
        mean = jnp.mean(h, axis=0, keepdims=True)
        ctr = h - mean
        var = jnp.mean(ctr * ctr, axis=0, keepdims=True)
        o_ref[...] = g_ref[...] * ctr * lax.rsqrt(var + BN_EPS) + bt_ref[...]

    return pl.pallas_call(
        body,
        out_shape=jax.ShapeDtypeStruct((N_NODES, D), jnp.float32),
    )(partials, W, b, gamma, beta)


def kernel(g, feats, W, b, gamma, beta):
    g4 = g.reshape(2, NW, CHUNKS, CH)
    partials = _sc_aggregate(g4, feats)
    return _tc_tail(partials, W, b.reshape(1, D),
                    gamma.reshape(1, D), beta.reshape(1, D))
